# baseline (speedup 1.0000x reference)
import functools

import jax
import jax.numpy as jnp
from jax import lax
from jax.experimental import pallas as pl
from jax.experimental.pallas import tpu as pltpu
from jax.experimental.pallas import tpu_sc as plsc

N = 10000
F = 128
NCLS = 919
E = 320000

NC = 2
NS = 16
NW = NC * NS
EPW = E // NW
CH = 125
NH = 2
CPH = EPW // CH // NH
NPAIRH = CPH // 2
SUB = 624
ZC = 48

@functools.cache
def _make_sc_agg():
    mesh = plsc.VectorSubcoreMesh(
        core_axis_name="c", subcore_axis_name="s",
        num_cores=NC, num_subcores=NS)
    return pl.kernel(
        _sc_agg_body,
        out_type=jax.ShapeDtypeStruct((NC, N, F), jnp.float32),
        mesh=mesh,
        scratch_types=[
            pltpu.VMEM((CPH, CH), jnp.int32),
            pltpu.VMEM((CPH, CH), jnp.int32),
            pltpu.VMEM((CH, F), jnp.float32),
            pltpu.VMEM((CH, F), jnp.float32),
            pltpu.VMEM_SHARED((N, F), jnp.float32),
            pltpu.SemaphoreType.DMA,
            pltpu.SemaphoreType.DMA,
            pltpu.SemaphoreType.DMA,
            pltpu.SemaphoreType.DMA,
        ],
    )


def _sc_agg(support, edges5):
    return _make_sc_agg()(support, edges5)


def _sc_agg_body(support, edges5, out, src_v, dst_v, rows_a, rows_b,
                 acc_sh, sem_a, sem_b, sem_sa, sem_sb):
    c = lax.axis_index("c")
    s = lax.axis_index("s")
    wid = s * NC + c

    def _zrow(r, _):
        for j in range(F // 16):
            rows_a[r, pl.ds(j * 16, 16)] = jnp.zeros((16,), jnp.float32)
        return 0

    lax.fori_loop(0, ZC, _zrow, 0)

    if True:
        for t in range(SUB // ZC):
            pltpu.sync_copy(rows_a.at[pl.ds(0, ZC)],
                            acc_sh.at[pl.ds(s * SUB + t * ZC, ZC)])

        @pl.when(s == NS - 1)
        def _():
            pltpu.sync_copy(rows_a.at[pl.ds(0, 16)],
                            acc_sh.at[pl.ds(NS * SUB, 16)])

        plsc.subcore_barrier()

        for h in range(NH):
            pltpu.sync_copy(edges5.at[0, wid, h], src_v)
            pltpu.sync_copy(edges5.at[1, wid, h], dst_v)

            pltpu.async_copy(support.at[src_v.at[0]], rows_a, sem_a)
            pltpu.async_copy(support.at[src_v.at[1]], rows_b, sem_b)

            def _step(i, _):
                c0 = i * 2
                pltpu.make_async_copy(
                    support.at[src_v.at[c0]], rows_a, sem_a).wait()
                pltpu.async_copy(rows_a, acc_sh.at[dst_v.at[c0]], sem_sa,
                                 add=True)
                pltpu.make_async_copy(
                    support.at[src_v.at[c0 + 1]], rows_b, sem_b).wait()
                pltpu.async_copy(rows_b, acc_sh.at[dst_v.at[c0 + 1]], sem_sb,
                                 add=True)
                pltpu.make_async_copy(
                    rows_a, acc_sh.at[dst_v.at[c0]], sem_sa).wait()
                pltpu.make_async_copy(
                    rows_b, acc_sh.at[dst_v.at[c0 + 1]], sem_sb).wait()

                @pl.when(i < NPAIRH - 1)
                def _():
                    pltpu.async_copy(
                        support.at[src_v.at[c0 + 2]], rows_a, sem_a)
                    pltpu.async_copy(
                        support.at[src_v.at[c0 + 3]], rows_b, sem_b)

                return 0

            lax.fori_loop(0, NPAIRH, _step, 0)

        plsc.subcore_barrier()
        pltpu.sync_copy(acc_sh.at[pl.ds(s * SUB, SUB)],
                        out.at[c, pl.ds(s * SUB, SUB)])

        @pl.when(s == NS - 1)
        def _():
            pltpu.sync_copy(acc_sh.at[pl.ds(NS * SUB, 16)],
                            out.at[c, pl.ds(NS * SUB, 16)])


def _bf16_dot(x, w):
    return jnp.dot(x.astype(jnp.bfloat16), w.astype(jnp.bfloat16),
                   preferred_element_type=jnp.float32)


def _mm_body(x_ref, w_ref, o_ref):
    o_ref[...] = _bf16_dot(x_ref[...], w_ref[...])


def _first_matmul(x, w, bm=2000):
    m, k = x.shape
    n = w.shape[1]
    return pl.pallas_call(
        _mm_body,
        grid=(m // bm,),
        in_specs=[pl.BlockSpec((bm, k), lambda i: (i, 0)),
                  pl.BlockSpec((k, n), lambda i: (0, 0))],
        out_specs=pl.BlockSpec((bm, n), lambda i: (i, 0)),
        out_shape=jax.ShapeDtypeStruct((m, n), jnp.float32),
    )(x, w)


def _mid_body(part_ref, deg_ref, b1_ref, wg1_ref, bg1_ref, x_ref, w2_ref,
              s2_ref, x1_ref, g_ref):
    agg = part_ref[0] + part_ref[1]
    z = jnp.tanh(agg / deg_ref[...] + b1_ref[...])
    g = jax.nn.sigmoid(
        jnp.sum(z * wg1_ref[...], axis=1, keepdims=True) + bg1_ref[...])
    x1 = (1.0 - g) * x_ref[...] + g * z
    s2_ref[...] = _bf16_dot(x1, w2_ref[...])
    x1_ref[...] = x1
    g_ref[...] = g


def _mid(part, deg, b1, wg1, bg1, x_in, w2, bm=2000):
    grid = (N // bm,)
    return pl.pallas_call(
        _mid_body,
        grid=grid,
        in_specs=[
            pl.BlockSpec((NC, bm, F), lambda i: (0, i, 0)),
            pl.BlockSpec((bm, 1), lambda i: (i, 0)),
            pl.BlockSpec((1, F), lambda i: (0, 0)),
            pl.BlockSpec((1, F), lambda i: (0, 0)),
            pl.BlockSpec((1, 1), lambda i: (0, 0)),
            pl.BlockSpec((bm, F), lambda i: (i, 0)),
            pl.BlockSpec((F, F), lambda i: (0, 0)),
        ],
        out_specs=[
            pl.BlockSpec((bm, F), lambda i: (i, 0)),
            pl.BlockSpec((bm, F), lambda i: (i, 0)),
            pl.BlockSpec((bm, 1), lambda i: (i, 0)),
        ],
        out_shape=[
            jax.ShapeDtypeStruct((N, F), jnp.float32),
            jax.ShapeDtypeStruct((N, F), jnp.float32),
            jax.ShapeDtypeStruct((N, 1), jnp.float32),
        ],
    )(part, deg, b1, wg1, bg1, x_in, w2)


def _final_body(part_ref, deg_ref, b2_ref, wg2_ref, bg2_ref, x1_ref,
                gam_ref, bet_ref, mu_ref, var_ref, wout_t_ref, bout_ref,
                out_ref, g2_ref):
    agg = part_ref[0] + part_ref[1]
    z2 = jnp.tanh(agg / deg_ref[...] + b2_ref[...])
    g2 = jax.nn.sigmoid(
        jnp.sum(z2 * wg2_ref[...], axis=1, keepdims=True) + bg2_ref[...])
    x = jax.nn.relu((1.0 - g2) * x1_ref[...] + g2 * z2)
    x = (x - mu_ref[...]) * jax.lax.rsqrt(var_ref[...] + 1e-5) \
        * gam_ref[...] + bet_ref[...]
    out_ref[...] = jax.lax.dot_general(
        wout_t_ref[...].astype(jnp.bfloat16), x.astype(jnp.bfloat16),
        dimension_numbers=(((1,), (1,)), ((), ())),
        preferred_element_type=jnp.float32) + bout_ref[...]
    g2_ref[...] = g2


def _final(part, deg, b2, wg2, bg2, x1, gam, bet, mu, var, wout, bout,
           bm=1280):
    grid = ((N + bm - 1) // bm,)
    vec = pl.BlockSpec((1, F), lambda i: (0, 0))
    return pl.pallas_call(
        _final_body,
        grid=grid,
        in_specs=[
            pl.BlockSpec((NC, bm, F), lambda i: (0, i, 0)),
            pl.BlockSpec((bm, 1), lambda i: (i, 0)),
            vec,
            vec,
            pl.BlockSpec((1, 1), lambda i: (0, 0)),
            pl.BlockSpec((bm, F), lambda i: (i, 0)),
            vec, vec, vec, vec,
            pl.BlockSpec((NCLS, F), lambda i: (0, 0)),
            pl.BlockSpec((NCLS, 1), lambda i: (0, 0)),
        ],
        out_specs=[
            pl.BlockSpec((NCLS, bm), lambda i: (0, i)),
            pl.BlockSpec((bm, 1), lambda i: (i, 0)),
        ],
        out_shape=[
            jax.ShapeDtypeStruct((NCLS, N), jnp.float32),
            jax.ShapeDtypeStruct((N, 1), jnp.float32),
        ],
    )(part, deg, b2, wg2, bg2, x1, gam, bet, mu, var, wout, bout)


def kernel(x_in, edge_index, deg, W_gc1, b_gc1, w_g1, b_g1, W_gc2, b_gc2,
           w_g2, b_g2, bn_gamma, bn_beta, bn_mean, bn_var, W_out, b_out):
    edges5 = edge_index.reshape(2, NW, NH, CPH, CH)

    support1 = _first_matmul(x_in, W_gc1)
    part1 = _sc_agg(support1, edges5)
    support2, x1, g = _mid(part1, deg, b_gc1.reshape(1, F),
                           w_g1.reshape(F)[None, :], b_g1.reshape(1, 1),
                           x_in, W_gc2)
    part2 = _sc_agg(support2, edges5)
    out_t, g2 = _final(part2, deg, b_gc2.reshape(1, F),
                       w_g2.reshape(F)[None, :], b_g2.reshape(1, 1), x1,
                       bn_gamma[None, :], bn_beta[None, :],
                       bn_mean[None, :], bn_var[None, :],
                       W_out.T, b_out[:, None])
    return (x_in, out_t.T, g, g2)


# speedup vs baseline: 1.2636x; 1.2636x over previous
import functools

import jax
import jax.numpy as jnp
from jax import lax
from jax.experimental import pallas as pl
from jax.experimental.pallas import tpu as pltpu
from jax.experimental.pallas import tpu_sc as plsc

N = 10000
F = 128
NCLS = 919
E = 320000

NC = 2
NS = 16
NW = NC * NS
EPW = E // NW
CH = 125
NH = 2
CPH = EPW // CH // NH
NPAIRH = CPH // 2
SUB = 624
ZC = 48

@functools.cache
def _make_sc_agg():
    mesh = plsc.VectorSubcoreMesh(
        core_axis_name="c", subcore_axis_name="s",
        num_cores=NC, num_subcores=NS)
    return pl.kernel(
        _sc_agg_body,
        out_type=jax.ShapeDtypeStruct((NC, N, F), jnp.float32),
        mesh=mesh,
        scratch_types=[
            pltpu.VMEM((CPH, CH), jnp.int32),
            pltpu.VMEM((CPH, CH), jnp.int32),
            pltpu.VMEM((CH, F), jnp.float32),
            pltpu.VMEM((CH, F), jnp.float32),
            pltpu.VMEM_SHARED((N, F), jnp.float32),
            pltpu.SemaphoreType.DMA,
            pltpu.SemaphoreType.DMA,
        ],
    )


def _sc_agg(support, edges5):
    return _make_sc_agg()(support, edges5)


def _sc_agg_body(support, edges5, out, src_v, dst_v, rows_a, rows_b,
                 acc_sh, sem_a, sem_b):
    c = lax.axis_index("c")
    s = lax.axis_index("s")
    wid = s * NC + c

    def _zrow(r, _):
        for j in range(F // 16):
            rows_a[r, pl.ds(j * 16, 16)] = jnp.zeros((16,), jnp.float32)
        return 0

    lax.fori_loop(0, ZC, _zrow, 0)

    if True:
        for t in range(SUB // ZC):
            pltpu.sync_copy(rows_a.at[pl.ds(0, ZC)],
                            acc_sh.at[pl.ds(s * SUB + t * ZC, ZC)])

        @pl.when(s == NS - 1)
        def _():
            pltpu.sync_copy(rows_a.at[pl.ds(0, 16)],
                            acc_sh.at[pl.ds(NS * SUB, 16)])

        plsc.subcore_barrier()

        for h in range(NH):
            pltpu.sync_copy(edges5.at[0, wid, h], src_v)
            pltpu.sync_copy(edges5.at[1, wid, h], dst_v)

            pltpu.async_copy(support.at[src_v.at[0]], rows_a, sem_a)

            def _step(i, _):
                c0 = i * 2
                pltpu.async_copy(support.at[src_v.at[c0 + 1]], rows_b, sem_b)
                pltpu.make_async_copy(
                    support.at[src_v.at[c0]], rows_a, sem_a).wait()
                pltpu.sync_copy(rows_a, acc_sh.at[dst_v.at[c0]], add=True)

                @pl.when(i < NPAIRH - 1)
                def _():
                    pltpu.async_copy(
                        support.at[src_v.at[c0 + 2]], rows_a, sem_a)

                pltpu.make_async_copy(
                    support.at[src_v.at[c0 + 1]], rows_b, sem_b).wait()
                pltpu.sync_copy(rows_b, acc_sh.at[dst_v.at[c0 + 1]], add=True)
                return 0

            lax.fori_loop(0, NPAIRH, _step, 0)

        plsc.subcore_barrier()
        pltpu.sync_copy(acc_sh.at[pl.ds(s * SUB, SUB)],
                        out.at[c, pl.ds(s * SUB, SUB)])

        @pl.when(s == NS - 1)
        def _():
            pltpu.sync_copy(acc_sh.at[pl.ds(NS * SUB, 16)],
                            out.at[c, pl.ds(NS * SUB, 16)])


def _bf16_dot(x, w):
    return jnp.dot(x.astype(jnp.bfloat16), w.astype(jnp.bfloat16),
                   preferred_element_type=jnp.float32)


def _mm_body(x_ref, w_ref, o_ref):
    o_ref[...] = _bf16_dot(x_ref[...], w_ref[...])


def _first_matmul(x, w, bm=2000):
    m, k = x.shape
    n = w.shape[1]
    return pl.pallas_call(
        _mm_body,
        grid=(m // bm,),
        in_specs=[pl.BlockSpec((bm, k), lambda i: (i, 0)),
                  pl.BlockSpec((k, n), lambda i: (0, 0))],
        out_specs=pl.BlockSpec((bm, n), lambda i: (i, 0)),
        out_shape=jax.ShapeDtypeStruct((m, n), jnp.float32),
    )(x, w)


def _mid_body(part_ref, deg_ref, b1_ref, wg1_ref, bg1_ref, x_ref, w2_ref,
              s2_ref, x1_ref, g_ref):
    agg = part_ref[0] + part_ref[1]
    z = jnp.tanh(agg / deg_ref[...] + b1_ref[...])
    g = jax.nn.sigmoid(
        jnp.sum(z * wg1_ref[...], axis=1, keepdims=True) + bg1_ref[...])
    x1 = (1.0 - g) * x_ref[...] + g * z
    s2_ref[...] = _bf16_dot(x1, w2_ref[...])
    x1_ref[...] = x1
    g_ref[...] = g


def _mid(part, deg, b1, wg1, bg1, x_in, w2, bm=2000):
    grid = (N // bm,)
    return pl.pallas_call(
        _mid_body,
        grid=grid,
        in_specs=[
            pl.BlockSpec((NC, bm, F), lambda i: (0, i, 0)),
            pl.BlockSpec((bm, 1), lambda i: (i, 0)),
            pl.BlockSpec((1, F), lambda i: (0, 0)),
            pl.BlockSpec((1, F), lambda i: (0, 0)),
            pl.BlockSpec((1, 1), lambda i: (0, 0)),
            pl.BlockSpec((bm, F), lambda i: (i, 0)),
            pl.BlockSpec((F, F), lambda i: (0, 0)),
        ],
        out_specs=[
            pl.BlockSpec((bm, F), lambda i: (i, 0)),
            pl.BlockSpec((bm, F), lambda i: (i, 0)),
            pl.BlockSpec((bm, 1), lambda i: (i, 0)),
        ],
        out_shape=[
            jax.ShapeDtypeStruct((N, F), jnp.float32),
            jax.ShapeDtypeStruct((N, F), jnp.float32),
            jax.ShapeDtypeStruct((N, 1), jnp.float32),
        ],
    )(part, deg, b1, wg1, bg1, x_in, w2)


def _final_body(part_ref, deg_ref, b2_ref, wg2_ref, bg2_ref, x1_ref,
                gam_ref, bet_ref, mu_ref, var_ref, wout_t_ref, bout_ref,
                out_ref, g2_ref):
    agg = part_ref[0] + part_ref[1]
    z2 = jnp.tanh(agg / deg_ref[...] + b2_ref[...])
    g2 = jax.nn.sigmoid(
        jnp.sum(z2 * wg2_ref[...], axis=1, keepdims=True) + bg2_ref[...])
    x = jax.nn.relu((1.0 - g2) * x1_ref[...] + g2 * z2)
    x = (x - mu_ref[...]) * jax.lax.rsqrt(var_ref[...] + 1e-5) \
        * gam_ref[...] + bet_ref[...]
    out_ref[...] = jax.lax.dot_general(
        wout_t_ref[...].astype(jnp.bfloat16), x.astype(jnp.bfloat16),
        dimension_numbers=(((1,), (1,)), ((), ())),
        preferred_element_type=jnp.float32) + bout_ref[...]
    g2_ref[...] = g2


def _final(part, deg, b2, wg2, bg2, x1, gam, bet, mu, var, wout, bout,
           bm=1280):
    grid = ((N + bm - 1) // bm,)
    vec = pl.BlockSpec((1, F), lambda i: (0, 0))
    return pl.pallas_call(
        _final_body,
        grid=grid,
        in_specs=[
            pl.BlockSpec((NC, bm, F), lambda i: (0, i, 0)),
            pl.BlockSpec((bm, 1), lambda i: (i, 0)),
            vec,
            vec,
            pl.BlockSpec((1, 1), lambda i: (0, 0)),
            pl.BlockSpec((bm, F), lambda i: (i, 0)),
            vec, vec, vec, vec,
            pl.BlockSpec((NCLS, F), lambda i: (0, 0)),
            pl.BlockSpec((NCLS, 1), lambda i: (0, 0)),
        ],
        out_specs=[
            pl.BlockSpec((NCLS, bm), lambda i: (0, i)),
            pl.BlockSpec((bm, 1), lambda i: (i, 0)),
        ],
        out_shape=[
            jax.ShapeDtypeStruct((NCLS, N), jnp.float32),
            jax.ShapeDtypeStruct((N, 1), jnp.float32),
        ],
    )(part, deg, b2, wg2, bg2, x1, gam, bet, mu, var, wout, bout)


def kernel(x_in, edge_index, deg, W_gc1, b_gc1, w_g1, b_g1, W_gc2, b_gc2,
           w_g2, b_g2, bn_gamma, bn_beta, bn_mean, bn_var, W_out, b_out):
    edges5 = edge_index.reshape(2, NW, NH, CPH, CH)

    support1 = _first_matmul(x_in, W_gc1)
    part1 = _sc_agg(support1, edges5)
    support2, x1, g = _mid(part1, deg, b_gc1.reshape(1, F),
                           w_g1.reshape(F)[None, :], b_g1.reshape(1, 1),
                           x_in, W_gc2)
    part2 = _sc_agg(support2, edges5)
    out_t, g2 = _final(part2, deg, b_gc2.reshape(1, F),
                       w_g2.reshape(F)[None, :], b_g2.reshape(1, 1), x1,
                       bn_gamma[None, :], bn_beta[None, :],
                       bn_mean[None, :], bn_var[None, :],
                       W_out.T, b_out[:, None])
    return (x_in, out_t.T, g, g2)


# speedup vs baseline: 1.2796x; 1.0127x over previous
import functools

import jax
import jax.numpy as jnp
from jax import lax
from jax.experimental import pallas as pl
from jax.experimental.pallas import tpu as pltpu
from jax.experimental.pallas import tpu_sc as plsc

N = 10000
F = 128
NCLS = 919
E = 320000

NC = 2
NS = 16
NW = NC * NS
EPW = E // NW
CH = 125
NH = 2
CPH = EPW // CH // NH
NPAIRH = CPH // 2
SUB = 624
ZC = 48

@functools.cache
def _make_sc_agg():
    mesh = plsc.VectorSubcoreMesh(
        core_axis_name="c", subcore_axis_name="s",
        num_cores=NC, num_subcores=NS)
    return pl.kernel(
        _sc_agg_body,
        out_type=jax.ShapeDtypeStruct((NC, N, F), jnp.float32),
        mesh=mesh,
        scratch_types=[
            pltpu.VMEM((CPH, CH), jnp.int32),
            pltpu.VMEM((CPH, CH), jnp.int32),
            pltpu.VMEM((CH, F), jnp.float32),
            pltpu.VMEM((CH, F), jnp.float32),
            pltpu.VMEM_SHARED((N, F), jnp.float32),
            pltpu.SemaphoreType.DMA,
            pltpu.SemaphoreType.DMA,
        ],
    )


def _sc_agg(support, edges5):
    return _make_sc_agg()(support, edges5)


def _sc_agg_body(support, edges5, out, src_v, dst_v, rows_a, rows_b,
                 acc_sh, sem_a, sem_b):
    c = lax.axis_index("c")
    s = lax.axis_index("s")
    wid = s * NC + c

    def _zrow(r, _):
        for j in range(F // 16):
            rows_a[r, pl.ds(j * 16, 16)] = jnp.zeros((16,), jnp.float32)
        return 0

    lax.fori_loop(0, ZC, _zrow, 0)

    if True:
        for t in range(SUB // ZC):
            pltpu.sync_copy(rows_a.at[pl.ds(0, ZC)],
                            acc_sh.at[pl.ds(s * SUB + t * ZC, ZC)])

        @pl.when(s == NS - 1)
        def _():
            pltpu.sync_copy(rows_a.at[pl.ds(0, 16)],
                            acc_sh.at[pl.ds(NS * SUB, 16)])

        plsc.subcore_barrier()

        for h in range(NH):
            pltpu.sync_copy(edges5.at[0, wid, h], src_v)
            pltpu.sync_copy(edges5.at[1, wid, h], dst_v)

            pltpu.async_copy(support.at[src_v.at[0]], rows_a, sem_a)

            def _step(i, _):
                c0 = i * 2
                pltpu.async_copy(support.at[src_v.at[c0 + 1]], rows_b, sem_b)
                pltpu.make_async_copy(
                    support.at[src_v.at[c0]], rows_a, sem_a).wait()
                pltpu.sync_copy(rows_a, acc_sh.at[dst_v.at[c0]], add=True)

                @pl.when(i < NPAIRH - 1)
                def _():
                    pltpu.async_copy(
                        support.at[src_v.at[c0 + 2]], rows_a, sem_a)

                pltpu.make_async_copy(
                    support.at[src_v.at[c0 + 1]], rows_b, sem_b).wait()
                pltpu.sync_copy(rows_b, acc_sh.at[dst_v.at[c0 + 1]], add=True)
                return 0

            lax.fori_loop(0, NPAIRH, _step, 0)

        plsc.subcore_barrier()
        pltpu.sync_copy(acc_sh.at[pl.ds(s * SUB, SUB)],
                        out.at[c, pl.ds(s * SUB, SUB)])

        @pl.when(s == NS - 1)
        def _():
            pltpu.sync_copy(acc_sh.at[pl.ds(NS * SUB, 16)],
                            out.at[c, pl.ds(NS * SUB, 16)])


def _bf16_dot(x, w):
    return jnp.dot(x.astype(jnp.bfloat16), w.astype(jnp.bfloat16),
                   preferred_element_type=jnp.float32)


def _mm_body(x_ref, w_ref, o_ref):
    o_ref[...] = _bf16_dot(x_ref[...], w_ref[...])


def _first_matmul(x, w, bm=2000):
    m, k = x.shape
    n = w.shape[1]
    return pl.pallas_call(
        _mm_body,
        grid=(m // bm,),
        in_specs=[pl.BlockSpec((bm, k), lambda i: (i, 0)),
                  pl.BlockSpec((k, n), lambda i: (0, 0))],
        out_specs=pl.BlockSpec((bm, n), lambda i: (i, 0)),
        out_shape=jax.ShapeDtypeStruct((m, n), jnp.float32),
    )(x, w)


def _mid_body(part_ref, deg_ref, b1_ref, wg1_ref, bg1_ref, x_ref, w2_ref,
              s2_ref, x1_ref, g_ref):
    agg = part_ref[0] + part_ref[1]
    z = jnp.tanh(agg / deg_ref[...] + b1_ref[...])
    g = jax.nn.sigmoid(
        jnp.sum(z * wg1_ref[...], axis=1, keepdims=True) + bg1_ref[...])
    x1 = (1.0 - g) * x_ref[...] + g * z
    s2_ref[...] = _bf16_dot(x1, w2_ref[...])
    x1_ref[...] = x1
    g_ref[...] = g


def _mid(part, deg, b1, wg1, bg1, x_in, w2, bm=5000):
    grid = (N // bm,)
    return pl.pallas_call(
        _mid_body,
        grid=grid,
        in_specs=[
            pl.BlockSpec((NC, bm, F), lambda i: (0, i, 0)),
            pl.BlockSpec((bm, 1), lambda i: (i, 0)),
            pl.BlockSpec((1, F), lambda i: (0, 0)),
            pl.BlockSpec((1, F), lambda i: (0, 0)),
            pl.BlockSpec((1, 1), lambda i: (0, 0)),
            pl.BlockSpec((bm, F), lambda i: (i, 0)),
            pl.BlockSpec((F, F), lambda i: (0, 0)),
        ],
        out_specs=[
            pl.BlockSpec((bm, F), lambda i: (i, 0)),
            pl.BlockSpec((bm, F), lambda i: (i, 0)),
            pl.BlockSpec((bm, 1), lambda i: (i, 0)),
        ],
        out_shape=[
            jax.ShapeDtypeStruct((N, F), jnp.float32),
            jax.ShapeDtypeStruct((N, F), jnp.float32),
            jax.ShapeDtypeStruct((N, 1), jnp.float32),
        ],
    )(part, deg, b1, wg1, bg1, x_in, w2)


def _final_body(part_ref, deg_ref, b2_ref, wg2_ref, bg2_ref, x1_ref,
                gam_ref, bet_ref, mu_ref, var_ref, wout_t_ref, bout_ref,
                out_ref, g2_ref):
    agg = part_ref[0] + part_ref[1]
    z2 = jnp.tanh(agg / deg_ref[...] + b2_ref[...])
    g2 = jax.nn.sigmoid(
        jnp.sum(z2 * wg2_ref[...], axis=1, keepdims=True) + bg2_ref[...])
    x = jax.nn.relu((1.0 - g2) * x1_ref[...] + g2 * z2)
    x = (x - mu_ref[...]) * jax.lax.rsqrt(var_ref[...] + 1e-5) \
        * gam_ref[...] + bet_ref[...]
    out_ref[...] = jax.lax.dot_general(
        wout_t_ref[...].astype(jnp.bfloat16), x.astype(jnp.bfloat16),
        dimension_numbers=(((1,), (1,)), ((), ())),
        preferred_element_type=jnp.float32) + bout_ref[...]
    g2_ref[...] = g2


def _final(part, deg, b2, wg2, bg2, x1, gam, bet, mu, var, wout, bout,
           bm=2560):
    grid = ((N + bm - 1) // bm,)
    vec = pl.BlockSpec((1, F), lambda i: (0, 0))
    return pl.pallas_call(
        _final_body,
        grid=grid,
        in_specs=[
            pl.BlockSpec((NC, bm, F), lambda i: (0, i, 0)),
            pl.BlockSpec((bm, 1), lambda i: (i, 0)),
            vec,
            vec,
            pl.BlockSpec((1, 1), lambda i: (0, 0)),
            pl.BlockSpec((bm, F), lambda i: (i, 0)),
            vec, vec, vec, vec,
            pl.BlockSpec((NCLS, F), lambda i: (0, 0)),
            pl.BlockSpec((NCLS, 1), lambda i: (0, 0)),
        ],
        out_specs=[
            pl.BlockSpec((NCLS, bm), lambda i: (0, i)),
            pl.BlockSpec((bm, 1), lambda i: (i, 0)),
        ],
        out_shape=[
            jax.ShapeDtypeStruct((NCLS, N), jnp.float32),
            jax.ShapeDtypeStruct((N, 1), jnp.float32),
        ],
    )(part, deg, b2, wg2, bg2, x1, gam, bet, mu, var, wout, bout)


def kernel(x_in, edge_index, deg, W_gc1, b_gc1, w_g1, b_g1, W_gc2, b_gc2,
           w_g2, b_g2, bn_gamma, bn_beta, bn_mean, bn_var, W_out, b_out):
    edges5 = edge_index.reshape(2, NW, NH, CPH, CH)

    support1 = _first_matmul(x_in, W_gc1)
    part1 = _sc_agg(support1, edges5)
    support2, x1, g = _mid(part1, deg, b_gc1.reshape(1, F),
                           w_g1.reshape(F)[None, :], b_g1.reshape(1, 1),
                           x_in, W_gc2)
    part2 = _sc_agg(support2, edges5)
    out_t, g2 = _final(part2, deg, b_gc2.reshape(1, F),
                       w_g2.reshape(F)[None, :], b_g2.reshape(1, 1), x1,
                       bn_gamma[None, :], bn_beta[None, :],
                       bn_mean[None, :], bn_var[None, :],
                       W_out.T, b_out[:, None])
    return (x_in, out_t.T, g, g2)


# speedup vs baseline: 1.2863x; 1.0052x over previous
import functools

import jax
import jax.numpy as jnp
from jax import lax
from jax.experimental import pallas as pl
from jax.experimental.pallas import tpu as pltpu
from jax.experimental.pallas import tpu_sc as plsc

N = 10000
F = 128
NCLS = 919
E = 320000

NC = 2
NS = 16
NW = NC * NS
EPW = E // NW
CH = 125
NH = 2
CPH = EPW // CH // NH
NPAIRH = CPH // 2
SUB = 624
ZC = 48

@functools.cache
def _make_sc_agg():
    mesh = plsc.VectorSubcoreMesh(
        core_axis_name="c", subcore_axis_name="s",
        num_cores=NC, num_subcores=NS)
    return pl.kernel(
        _sc_agg_body,
        out_type=jax.ShapeDtypeStruct((NC, N, F), jnp.float32),
        mesh=mesh,
        scratch_types=[
            pltpu.VMEM((CPH, CH), jnp.int32),
            pltpu.VMEM((CPH, CH), jnp.int32),
            pltpu.VMEM((CH, F), jnp.float32),
            pltpu.VMEM((CH, F), jnp.float32),
            pltpu.VMEM_SHARED((N, F), jnp.float32),
            pltpu.SemaphoreType.DMA,
            pltpu.SemaphoreType.DMA,
        ],
    )


def _sc_agg(support, edges5):
    return _make_sc_agg()(support, edges5)


def _sc_agg_body(support, edges5, out, src_v, dst_v, rows_a, rows_b,
                 acc_sh, sem_a, sem_b):
    c = lax.axis_index("c")
    s = lax.axis_index("s")
    wid = s * NC + c

    pltpu.sync_copy(edges5.at[0, wid, 0], src_v)
    pltpu.sync_copy(edges5.at[1, wid, 0], dst_v)
    pltpu.async_copy(support.at[src_v.at[0]], rows_b, sem_b)

    def _zrow(r, _):
        for j in range(F // 16):
            rows_a[r, pl.ds(j * 16, 16)] = jnp.zeros((16,), jnp.float32)
        return 0

    lax.fori_loop(0, ZC, _zrow, 0)

    if True:
        for t in range(SUB // ZC):
            pltpu.sync_copy(rows_a.at[pl.ds(0, ZC)],
                            acc_sh.at[pl.ds(s * SUB + t * ZC, ZC)])

        @pl.when(s == NS - 1)
        def _():
            pltpu.sync_copy(rows_a.at[pl.ds(0, 16)],
                            acc_sh.at[pl.ds(NS * SUB, 16)])

        plsc.subcore_barrier()

        for h in range(NH):
            if h > 0:
                pltpu.sync_copy(edges5.at[0, wid, h], src_v)
                pltpu.sync_copy(edges5.at[1, wid, h], dst_v)
                pltpu.async_copy(support.at[src_v.at[0]], rows_b, sem_b)

            def _step(i, _):
                c0 = i * 2
                pltpu.async_copy(support.at[src_v.at[c0 + 1]], rows_a, sem_a)
                pltpu.make_async_copy(
                    support.at[src_v.at[c0]], rows_b, sem_b).wait()
                pltpu.sync_copy(rows_b, acc_sh.at[dst_v.at[c0]], add=True)

                @pl.when(i < NPAIRH - 1)
                def _():
                    pltpu.async_copy(
                        support.at[src_v.at[c0 + 2]], rows_b, sem_b)

                pltpu.make_async_copy(
                    support.at[src_v.at[c0 + 1]], rows_a, sem_a).wait()
                pltpu.sync_copy(rows_a, acc_sh.at[dst_v.at[c0 + 1]], add=True)
                return 0

            lax.fori_loop(0, NPAIRH, _step, 0)

        plsc.subcore_barrier()
        pltpu.sync_copy(acc_sh.at[pl.ds(s * SUB, SUB)],
                        out.at[c, pl.ds(s * SUB, SUB)])

        @pl.when(s == NS - 1)
        def _():
            pltpu.sync_copy(acc_sh.at[pl.ds(NS * SUB, 16)],
                            out.at[c, pl.ds(NS * SUB, 16)])


def _bf16_dot(x, w):
    return jnp.dot(x.astype(jnp.bfloat16), w.astype(jnp.bfloat16),
                   preferred_element_type=jnp.float32)


def _mm_body(x_ref, w_ref, o_ref):
    o_ref[...] = _bf16_dot(x_ref[...], w_ref[...])


def _first_matmul(x, w, bm=2000):
    m, k = x.shape
    n = w.shape[1]
    return pl.pallas_call(
        _mm_body,
        grid=(m // bm,),
        in_specs=[pl.BlockSpec((bm, k), lambda i: (i, 0)),
                  pl.BlockSpec((k, n), lambda i: (0, 0))],
        out_specs=pl.BlockSpec((bm, n), lambda i: (i, 0)),
        out_shape=jax.ShapeDtypeStruct((m, n), jnp.float32),
    )(x, w)


def _mid_body(part_ref, deg_ref, b1_ref, wg1_ref, bg1_ref, x_ref, w2_ref,
              s2_ref, x1_ref, g_ref):
    agg = part_ref[0] + part_ref[1]
    z = jnp.tanh(agg / deg_ref[...] + b1_ref[...])
    g = jax.nn.sigmoid(
        jnp.sum(z * wg1_ref[...], axis=1, keepdims=True) + bg1_ref[...])
    x1 = (1.0 - g) * x_ref[...] + g * z
    s2_ref[...] = _bf16_dot(x1, w2_ref[...])
    x1_ref[...] = x1
    g_ref[...] = g


def _mid(part, deg, b1, wg1, bg1, x_in, w2, bm=5000):
    grid = (N // bm,)
    return pl.pallas_call(
        _mid_body,
        grid=grid,
        in_specs=[
            pl.BlockSpec((NC, bm, F), lambda i: (0, i, 0)),
            pl.BlockSpec((bm, 1), lambda i: (i, 0)),
            pl.BlockSpec((1, F), lambda i: (0, 0)),
            pl.BlockSpec((1, F), lambda i: (0, 0)),
            pl.BlockSpec((1, 1), lambda i: (0, 0)),
            pl.BlockSpec((bm, F), lambda i: (i, 0)),
            pl.BlockSpec((F, F), lambda i: (0, 0)),
        ],
        out_specs=[
            pl.BlockSpec((bm, F), lambda i: (i, 0)),
            pl.BlockSpec((bm, F), lambda i: (i, 0)),
            pl.BlockSpec((bm, 1), lambda i: (i, 0)),
        ],
        out_shape=[
            jax.ShapeDtypeStruct((N, F), jnp.float32),
            jax.ShapeDtypeStruct((N, F), jnp.float32),
            jax.ShapeDtypeStruct((N, 1), jnp.float32),
        ],
    )(part, deg, b1, wg1, bg1, x_in, w2)


def _final_body(part_ref, deg_ref, b2_ref, wg2_ref, bg2_ref, x1_ref,
                gam_ref, bet_ref, mu_ref, var_ref, wout_t_ref, bout_ref,
                out_ref, g2_ref):
    agg = part_ref[0] + part_ref[1]
    z2 = jnp.tanh(agg / deg_ref[...] + b2_ref[...])
    g2 = jax.nn.sigmoid(
        jnp.sum(z2 * wg2_ref[...], axis=1, keepdims=True) + bg2_ref[...])
    x = jax.nn.relu((1.0 - g2) * x1_ref[...] + g2 * z2)
    x = (x - mu_ref[...]) * jax.lax.rsqrt(var_ref[...] + 1e-5) \
        * gam_ref[...] + bet_ref[...]
    out_ref[...] = jax.lax.dot_general(
        wout_t_ref[...].astype(jnp.bfloat16), x.astype(jnp.bfloat16),
        dimension_numbers=(((1,), (1,)), ((), ())),
        preferred_element_type=jnp.float32) + bout_ref[...]
    g2_ref[...] = g2


def _final(part, deg, b2, wg2, bg2, x1, gam, bet, mu, var, wout, bout,
           bm=2560):
    grid = ((N + bm - 1) // bm,)
    vec = pl.BlockSpec((1, F), lambda i: (0, 0))
    return pl.pallas_call(
        _final_body,
        grid=grid,
        in_specs=[
            pl.BlockSpec((NC, bm, F), lambda i: (0, i, 0)),
            pl.BlockSpec((bm, 1), lambda i: (i, 0)),
            vec,
            vec,
            pl.BlockSpec((1, 1), lambda i: (0, 0)),
            pl.BlockSpec((bm, F), lambda i: (i, 0)),
            vec, vec, vec, vec,
            pl.BlockSpec((NCLS, F), lambda i: (0, 0)),
            pl.BlockSpec((NCLS, 1), lambda i: (0, 0)),
        ],
        out_specs=[
            pl.BlockSpec((NCLS, bm), lambda i: (0, i)),
            pl.BlockSpec((bm, 1), lambda i: (i, 0)),
        ],
        out_shape=[
            jax.ShapeDtypeStruct((NCLS, N), jnp.float32),
            jax.ShapeDtypeStruct((N, 1), jnp.float32),
        ],
    )(part, deg, b2, wg2, bg2, x1, gam, bet, mu, var, wout, bout)


def kernel(x_in, edge_index, deg, W_gc1, b_gc1, w_g1, b_g1, W_gc2, b_gc2,
           w_g2, b_g2, bn_gamma, bn_beta, bn_mean, bn_var, W_out, b_out):
    edges5 = edge_index.reshape(2, NW, NH, CPH, CH)

    support1 = _first_matmul(x_in, W_gc1)
    part1 = _sc_agg(support1, edges5)
    support2, x1, g = _mid(part1, deg, b_gc1.reshape(1, F),
                           w_g1.reshape(F)[None, :], b_g1.reshape(1, 1),
                           x_in, W_gc2)
    part2 = _sc_agg(support2, edges5)
    out_t, g2 = _final(part2, deg, b_gc2.reshape(1, F),
                       w_g2.reshape(F)[None, :], b_g2.reshape(1, 1), x1,
                       bn_gamma[None, :], bn_beta[None, :],
                       bn_mean[None, :], bn_var[None, :],
                       W_out.T, b_out[:, None])
    return (x_in, out_t.T, g, g2)


# speedup vs baseline: 1.2899x; 1.0028x over previous
import functools

import jax
import jax.numpy as jnp
from jax import lax
from jax.experimental import pallas as pl
from jax.experimental.pallas import tpu as pltpu
from jax.experimental.pallas import tpu_sc as plsc

N = 10000
F = 128
NCLS = 919
E = 320000

NC = 2
NS = 16
NW = NC * NS
EPW = E // NW
CH = 125
NH = 2
CPH = EPW // CH // NH
NPAIRH = CPH // 2
SUB = 624
ZC = 48

@functools.cache
def _make_sc_agg():
    mesh = plsc.VectorSubcoreMesh(
        core_axis_name="c", subcore_axis_name="s",
        num_cores=NC, num_subcores=NS)
    return pl.kernel(
        _sc_agg_body,
        out_type=jax.ShapeDtypeStruct((NC, N, F), jnp.float32),
        mesh=mesh,
        scratch_types=[
            pltpu.VMEM((CPH, CH), jnp.int32),
            pltpu.VMEM((CPH, CH), jnp.int32),
            pltpu.VMEM((CH, F), jnp.float32),
            pltpu.VMEM((CH, F), jnp.float32),
            pltpu.VMEM_SHARED((N, F), jnp.float32),
            pltpu.SemaphoreType.DMA,
            pltpu.SemaphoreType.DMA,
        ],
    )


def _sc_agg(support, edges5):
    return _make_sc_agg()(support, edges5)


def _sc_agg_body(support, edges5, out, src_v, dst_v, rows_a, rows_b,
                 acc_sh, sem_a, sem_b):
    c = lax.axis_index("c")
    s = lax.axis_index("s")
    wid = s * NC + c

    pltpu.sync_copy(edges5.at[0, wid, 0], src_v)
    pltpu.sync_copy(edges5.at[1, wid, 0], dst_v)
    pltpu.async_copy(support.at[src_v.at[0]], rows_b, sem_b)

    def _zrow(r, _):
        for j in range(F // 16):
            rows_a[r, pl.ds(j * 16, 16)] = jnp.zeros((16,), jnp.float32)
        return 0

    lax.fori_loop(0, ZC, _zrow, 0)

    if True:
        for t in range(SUB // ZC):
            pltpu.sync_copy(rows_a.at[pl.ds(0, ZC)],
                            acc_sh.at[pl.ds(s * SUB + t * ZC, ZC)])

        @pl.when(s == NS - 1)
        def _():
            pltpu.sync_copy(rows_a.at[pl.ds(0, 16)],
                            acc_sh.at[pl.ds(NS * SUB, 16)])

        plsc.subcore_barrier()

        for h in range(NH):
            if h > 0:
                pltpu.sync_copy(edges5.at[0, wid, h], src_v)
                pltpu.sync_copy(edges5.at[1, wid, h], dst_v)
                pltpu.async_copy(support.at[src_v.at[0]], rows_b, sem_b)

            def _step(i, _):
                c0 = i * 2
                pltpu.async_copy(support.at[src_v.at[c0 + 1]], rows_a, sem_a)
                pltpu.make_async_copy(
                    support.at[src_v.at[c0]], rows_b, sem_b).wait()
                pltpu.sync_copy(rows_b, acc_sh.at[dst_v.at[c0]], add=True)

                @pl.when(i < NPAIRH - 1)
                def _():
                    pltpu.async_copy(
                        support.at[src_v.at[c0 + 2]], rows_b, sem_b)

                pltpu.make_async_copy(
                    support.at[src_v.at[c0 + 1]], rows_a, sem_a).wait()
                pltpu.sync_copy(rows_a, acc_sh.at[dst_v.at[c0 + 1]], add=True)
                return 0

            lax.fori_loop(0, NPAIRH, _step, 0)

        plsc.subcore_barrier()
        pltpu.sync_copy(acc_sh.at[pl.ds(s * SUB, SUB)],
                        out.at[c, pl.ds(s * SUB, SUB)])

        @pl.when(s == NS - 1)
        def _():
            pltpu.sync_copy(acc_sh.at[pl.ds(NS * SUB, 16)],
                            out.at[c, pl.ds(NS * SUB, 16)])


def _bf16_dot(x, w):
    return jnp.dot(x.astype(jnp.bfloat16), w.astype(jnp.bfloat16),
                   preferred_element_type=jnp.float32)


def _mm_body(x_ref, w_ref, o_ref):
    o_ref[...] = _bf16_dot(x_ref[...], w_ref[...])


def _first_matmul(x, w, bm=10000):
    m, k = x.shape
    n = w.shape[1]
    return pl.pallas_call(
        _mm_body,
        grid=(m // bm,),
        in_specs=[pl.BlockSpec((bm, k), lambda i: (i, 0)),
                  pl.BlockSpec((k, n), lambda i: (0, 0))],
        out_specs=pl.BlockSpec((bm, n), lambda i: (i, 0)),
        out_shape=jax.ShapeDtypeStruct((m, n), jnp.float32),
    )(x, w)


def _mid_body(part_ref, deg_ref, b1_ref, wg1_ref, bg1_ref, x_ref, w2_ref,
              s2_ref, x1_ref, g_ref):
    agg = part_ref[0] + part_ref[1]
    z = jnp.tanh(agg / deg_ref[...] + b1_ref[...])
    g = jax.nn.sigmoid(
        jnp.sum(z * wg1_ref[...], axis=1, keepdims=True) + bg1_ref[...])
    x1 = (1.0 - g) * x_ref[...] + g * z
    s2_ref[...] = _bf16_dot(x1, w2_ref[...])
    x1_ref[...] = x1
    g_ref[...] = g


def _mid(part, deg, b1, wg1, bg1, x_in, w2, bm=5000):
    grid = (N // bm,)
    return pl.pallas_call(
        _mid_body,
        grid=grid,
        in_specs=[
            pl.BlockSpec((NC, bm, F), lambda i: (0, i, 0)),
            pl.BlockSpec((bm, 1), lambda i: (i, 0)),
            pl.BlockSpec((1, F), lambda i: (0, 0)),
            pl.BlockSpec((1, F), lambda i: (0, 0)),
            pl.BlockSpec((1, 1), lambda i: (0, 0)),
            pl.BlockSpec((bm, F), lambda i: (i, 0)),
            pl.BlockSpec((F, F), lambda i: (0, 0)),
        ],
        out_specs=[
            pl.BlockSpec((bm, F), lambda i: (i, 0)),
            pl.BlockSpec((bm, F), lambda i: (i, 0)),
            pl.BlockSpec((bm, 1), lambda i: (i, 0)),
        ],
        out_shape=[
            jax.ShapeDtypeStruct((N, F), jnp.float32),
            jax.ShapeDtypeStruct((N, F), jnp.float32),
            jax.ShapeDtypeStruct((N, 1), jnp.float32),
        ],
    )(part, deg, b1, wg1, bg1, x_in, w2)


def _final_body(part_ref, deg_ref, b2_ref, wg2_ref, bg2_ref, x1_ref,
                gam_ref, bet_ref, mu_ref, var_ref, wout_t_ref, bout_ref,
                out_ref, g2_ref):
    agg = part_ref[0] + part_ref[1]
    z2 = jnp.tanh(agg / deg_ref[...] + b2_ref[...])
    g2 = jax.nn.sigmoid(
        jnp.sum(z2 * wg2_ref[...], axis=1, keepdims=True) + bg2_ref[...])
    x = jax.nn.relu((1.0 - g2) * x1_ref[...] + g2 * z2)
    x = (x - mu_ref[...]) * jax.lax.rsqrt(var_ref[...] + 1e-5) \
        * gam_ref[...] + bet_ref[...]
    out_ref[...] = jax.lax.dot_general(
        wout_t_ref[...].astype(jnp.bfloat16), x.astype(jnp.bfloat16),
        dimension_numbers=(((1,), (1,)), ((), ())),
        preferred_element_type=jnp.float32) + bout_ref[...]
    g2_ref[...] = g2


def _final(part, deg, b2, wg2, bg2, x1, gam, bet, mu, var, wout, bout,
           bm=2560):
    grid = ((N + bm - 1) // bm,)
    vec = pl.BlockSpec((1, F), lambda i: (0, 0))
    return pl.pallas_call(
        _final_body,
        grid=grid,
        in_specs=[
            pl.BlockSpec((NC, bm, F), lambda i: (0, i, 0)),
            pl.BlockSpec((bm, 1), lambda i: (i, 0)),
            vec,
            vec,
            pl.BlockSpec((1, 1), lambda i: (0, 0)),
            pl.BlockSpec((bm, F), lambda i: (i, 0)),
            vec, vec, vec, vec,
            pl.BlockSpec((NCLS, F), lambda i: (0, 0)),
            pl.BlockSpec((NCLS, 1), lambda i: (0, 0)),
        ],
        out_specs=[
            pl.BlockSpec((NCLS, bm), lambda i: (0, i)),
            pl.BlockSpec((bm, 1), lambda i: (i, 0)),
        ],
        out_shape=[
            jax.ShapeDtypeStruct((NCLS, N), jnp.float32),
            jax.ShapeDtypeStruct((N, 1), jnp.float32),
        ],
    )(part, deg, b2, wg2, bg2, x1, gam, bet, mu, var, wout, bout)


def kernel(x_in, edge_index, deg, W_gc1, b_gc1, w_g1, b_g1, W_gc2, b_gc2,
           w_g2, b_g2, bn_gamma, bn_beta, bn_mean, bn_var, W_out, b_out):
    edges5 = edge_index.reshape(2, NW, NH, CPH, CH)

    support1 = _first_matmul(x_in, W_gc1)
    part1 = _sc_agg(support1, edges5)
    support2, x1, g = _mid(part1, deg, b_gc1.reshape(1, F),
                           w_g1.reshape(F)[None, :], b_g1.reshape(1, 1),
                           x_in, W_gc2)
    part2 = _sc_agg(support2, edges5)
    out_t, g2 = _final(part2, deg, b_gc2.reshape(1, F),
                       w_g2.reshape(F)[None, :], b_g2.reshape(1, 1), x1,
                       bn_gamma[None, :], bn_beta[None, :],
                       bn_mean[None, :], bn_var[None, :],
                       W_out.T, b_out[:, None])
    return (x_in, out_t.T, g, g2)
